# baseline (speedup 1.0000x reference)
import jax
import jax.numpy as jnp
from jax.experimental import pallas as pl


def _pos_kernel(w_ref, o_ref):
    h = w = 32
    cy, cx = h // 2, w // 2
    n = h * w
    ij = jax.lax.broadcasted_iota(jnp.int32, (1, n), 1)
    i = ij // w
    j = ij % w
    z = jnp.maximum(jnp.abs(cx - j) + jnp.abs(cy - i) - 1, 0)
    rows = jax.lax.broadcasted_iota(jnp.int32, (h, n), 0)
    onehot = (rows == z).astype(jnp.float32)
    out = jax.lax.dot_general(
        w_ref[...], onehot,
        dimension_numbers=(((0,), (0,)), ((), ())),
        preferred_element_type=jnp.float32,
    )
    o_ref[0, :, :] = out


def kernel(x, W):
    b = x.shape[0]
    nf = W.shape[1]
    h, w = x.shape[-2], x.shape[-1]
    n = h * w
    out = pl.pallas_call(
        _pos_kernel,
        grid=(b,),
        in_specs=[pl.BlockSpec((W.shape[0], nf), lambda i: (0, 0))],
        out_specs=pl.BlockSpec((1, nf, n), lambda i: (i, 0, 0)),
        out_shape=jax.ShapeDtypeStruct((b, nf, n), jnp.float32),
    )(W)
    return out.reshape(b, nf, h, w)


# speedup vs baseline: 1.0846x; 1.0846x over previous
import jax
import jax.numpy as jnp
from jax.experimental import pallas as pl
from jax.experimental.pallas import tpu as pltpu


def _pos_kernel(w_ref, o_ref):
    h = w = 32
    cy, cx = h // 2, w // 2
    n = h * w
    ij = jax.lax.broadcasted_iota(jnp.int32, (1, n), 1)
    i = ij // w
    j = ij % w
    z = jnp.maximum(jnp.abs(cx - j) + jnp.abs(cy - i) - 1, 0)
    rows = jax.lax.broadcasted_iota(jnp.int32, (h, n), 0)
    onehot = (rows == z).astype(jnp.float32)
    out = jax.lax.dot_general(
        w_ref[...], onehot,
        dimension_numbers=(((0,), (0,)), ((), ())),
        preferred_element_type=jnp.float32,
    )
    for bb in range(o_ref.shape[0]):
        o_ref[bb, :, :] = out


def kernel(x, W):
    b = x.shape[0]
    nf = W.shape[1]
    h, w = x.shape[-2], x.shape[-1]
    n = h * w
    bblk = 2
    out = pl.pallas_call(
        _pos_kernel,
        grid=(b // bblk,),
        in_specs=[pl.BlockSpec((W.shape[0], nf), lambda i: (0, 0))],
        out_specs=pl.BlockSpec((bblk, nf, n), lambda i: (i, 0, 0)),
        out_shape=jax.ShapeDtypeStruct((b, nf, n), jnp.float32),
        compiler_params=pltpu.CompilerParams(
            dimension_semantics=("parallel",),
        ),
    )(W)
    return out.reshape(b, nf, h, w)


# speedup vs baseline: 1.0917x; 1.0066x over previous
import jax
import jax.numpy as jnp
from jax.experimental import pallas as pl
from jax.experimental.pallas import tpu as pltpu


def _pos_kernel(w_ref, o_ref, tile_ref, sems):
    h = w = 32
    cy, cx = h // 2, w // 2
    n = h * w
    ij = jax.lax.broadcasted_iota(jnp.int32, (1, n), 1)
    i = ij // w
    j = ij % w
    z = jnp.maximum(jnp.abs(cx - j) + jnp.abs(cy - i) - 1, 0)
    rows = jax.lax.broadcasted_iota(jnp.int32, (h, n), 0)
    onehot = (rows == z).astype(jnp.float32)
    tile_ref[...] = jax.lax.dot_general(
        w_ref[...], onehot,
        dimension_numbers=(((0,), (0,)), ((), ())),
        preferred_element_type=jnp.float32,
    )
    nb = o_ref.shape[0]
    copies = [
        pltpu.make_async_copy(tile_ref, o_ref.at[b], sems.at[b])
        for b in range(nb)
    ]
    for c in copies:
        c.start()
    for c in copies:
        c.wait()


def kernel(x, W):
    b = x.shape[0]
    nf = W.shape[1]
    h, w = x.shape[-2], x.shape[-1]
    n = h * w
    out = pl.pallas_call(
        _pos_kernel,
        in_specs=[pl.BlockSpec(memory_space=pltpu.MemorySpace.VMEM)],
        out_specs=pl.BlockSpec(memory_space=pltpu.MemorySpace.HBM),
        out_shape=jax.ShapeDtypeStruct((b, nf, n), jnp.float32),
        scratch_shapes=[
            pltpu.MemorySpace.VMEM((nf, n), jnp.float32),
            pltpu.SemaphoreType.DMA((b,)),
        ],
    )(W)
    return out.reshape(b, nf, h, w)
